# baseline (speedup 1.0000x reference)
import functools

import jax
import jax.numpy as jnp
from jax import lax
from jax.experimental import pallas as pl
from jax.experimental.pallas import tpu as pltpu
from jax.experimental.pallas import tpu_sc as plsc

_N_FIELDS = 26
_VOCAB = 100000
_EDIM = 8
_BATCH = 16384

_INFO = plsc.get_sparse_core_info()
_NC = _INFO.num_cores
_NS = _INFO.num_subcores
_NW = _NC * _NS
_L = _INFO.num_lanes

_TASKS = _N_FIELDS * _EDIM
_QTR = _BATCH // 4


def _run_task(table_hbm, idx_hbm, out_hbm, slice_v, idx_v, outs, sems, t, q0, nq):
    f = t // _EDIM
    d = t % _EDIM
    pltpu.sync_copy(table_hbm.at[f, d], slice_v)
    cps = [None, None]
    for k in range(nq):
        b = k % 2
        if cps[b] is not None:
            cps[b].wait()
        qs = (q0 + k) * _QTR
        pltpu.sync_copy(idx_hbm.at[f, pl.ds(qs, _QTR)], idx_v)
        out_v = outs[b]

        @plsc.parallel_loop(0, _QTR // _L, 1, unroll=8)
        def gather(i):
            sl = pl.ds(i * _L, _L)
            out_v[sl] = plsc.load_gather(slice_v, [idx_v[sl]])

        cps[b] = pltpu.async_copy(out_v, out_hbm.at[t, pl.ds(qs, _QTR)], sems[b])
    for cp in cps:
        if cp is not None:
            cp.wait()


def _body(table_hbm, idx_hbm, out_hbm, slice_v, idx_v, out0_v, out1_v, sem0, sem1):
    wid = lax.axis_index("s") * _NC + lax.axis_index("c")
    outs, sems = (out0_v, out1_v), (sem0, sem1)

    for q in range(6):
        _run_task(table_hbm, idx_hbm, out_hbm, slice_v, idx_v, outs, sems,
                  q * _NW + wid, 0, 4)
    _run_task(table_hbm, idx_hbm, out_hbm, slice_v, idx_v, outs, sems,
              6 * _NW + lax.rem(wid, 16), lax.div(wid, 16) * 2, 2)


@functools.partial(
    pl.kernel,
    mesh=plsc.VectorSubcoreMesh(core_axis_name="c", subcore_axis_name="s"),
    out_type=jax.ShapeDtypeStruct((_N_FIELDS * _EDIM, _BATCH), jnp.float32),
    scratch_types=[
        pltpu.VMEM((_VOCAB,), jnp.float32),
        pltpu.VMEM((_QTR,), jnp.int32),
        pltpu.VMEM((_QTR,), jnp.float32),
        pltpu.VMEM((_QTR,), jnp.float32),
        pltpu.SemaphoreType.DMA,
        pltpu.SemaphoreType.DMA,
    ],
    compiler_params=pltpu.CompilerParams(
        use_tc_tiling_on_sc=True, needs_layout_passes=False
    ),
)
def _sc_gather(table_hbm, idx_hbm, out_hbm, slice_v, idx_v, out0_v, out1_v, s0, s1):
    _body(table_hbm, idx_hbm, out_hbm, slice_v, idx_v, out0_v, out1_v, s0, s1)


def kernel(inputs, tables):
    table_t = jnp.transpose(tables, (0, 2, 1))
    idx_t = inputs.T
    out_t = _sc_gather(table_t, idx_t)
    return out_t.T


# speedup vs baseline: 1.0931x; 1.0931x over previous
import functools

import jax
import jax.numpy as jnp
from jax import lax
from jax.experimental import pallas as pl
from jax.experimental.pallas import tpu as pltpu
from jax.experimental.pallas import tpu_sc as plsc

_N_FIELDS = 26
_VOCAB = 100000
_EDIM = 8
_BATCH = 16384

_INFO = plsc.get_sparse_core_info()
_NC = _INFO.num_cores
_NS = _INFO.num_subcores
_NW = _NC * _NS
_L = _INFO.num_lanes

_TASKS = _N_FIELDS * _EDIM
_HALF = _BATCH // 2


def _run_task(table_hbm, idx_hbm, out_hbm, slice_v, idx_v, out_v, t, halves):
    f = t // _EDIM
    d = t % _EDIM
    pltpu.sync_copy(table_hbm.at[f, d], slice_v)
    for h in halves:
        pltpu.sync_copy(idx_hbm.at[f, pl.ds(h * _HALF, _HALF)], idx_v)

        @plsc.parallel_loop(0, _HALF // _L, 1, unroll=8)
        def gather(i):
            sl = pl.ds(i * _L, _L)
            out_v[sl] = plsc.load_gather(slice_v, [idx_v[sl]])

        pltpu.sync_copy(out_v, out_hbm.at[t, pl.ds(h * _HALF, _HALF)])


def _body(table_hbm, idx_hbm, out_hbm, slice_v, idx_v, out_v):
    wid = lax.axis_index("s") * _NC + lax.axis_index("c")

    def round_body(q, _):
        _run_task(table_hbm, idx_hbm, out_hbm, slice_v, idx_v, out_v,
                  q * _NW + wid, (0, 1))
        return _

    lax.fori_loop(0, 6, round_body, None)
    _run_task(table_hbm, idx_hbm, out_hbm, slice_v, idx_v, out_v,
              6 * _NW + lax.rem(wid, 16), (lax.div(wid, 16),))


@functools.partial(
    pl.kernel,
    mesh=plsc.VectorSubcoreMesh(core_axis_name="c", subcore_axis_name="s"),
    out_type=jax.ShapeDtypeStruct((_N_FIELDS * _EDIM, _BATCH), jnp.float32),
    scratch_types=[
        pltpu.VMEM((_VOCAB,), jnp.float32),
        pltpu.VMEM((_HALF,), jnp.int32),
        pltpu.VMEM((_HALF,), jnp.float32),
    ],
    compiler_params=pltpu.CompilerParams(
        use_tc_tiling_on_sc=True, needs_layout_passes=False
    ),
)
def _sc_gather(table_hbm, idx_hbm, out_hbm, slice_v, idx_v, out_v):
    _body(table_hbm, idx_hbm, out_hbm, slice_v, idx_v, out_v)


def kernel(inputs, tables):
    table_t = jnp.transpose(tables, (0, 2, 1))
    idx_t = inputs.T
    out_t = _sc_gather(table_t, idx_t)
    return out_t.T


# speedup vs baseline: 1.0934x; 1.0003x over previous
import functools

import jax
import jax.numpy as jnp
from jax import lax
from jax.experimental import pallas as pl
from jax.experimental.pallas import tpu as pltpu
from jax.experimental.pallas import tpu_sc as plsc

_N_FIELDS = 26
_VOCAB = 100000
_EDIM = 8
_BATCH = 16384

_INFO = plsc.get_sparse_core_info()
_NC = _INFO.num_cores
_NS = _INFO.num_subcores
_NW = _NC * _NS
_L = _INFO.num_lanes

_TASKS = _N_FIELDS * _EDIM
_HALF = _BATCH // 2


def _run_task(table_hbm, idx_hbm, out_hbm, slice_v, idx_v, out_v, t, halves):
    f = t // _EDIM
    d = t % _EDIM
    pltpu.sync_copy(table_hbm.at[f, d], slice_v)
    for h in halves:
        pltpu.sync_copy(idx_hbm.at[f, pl.ds(h * _HALF, _HALF)], idx_v)

        @plsc.parallel_loop(0, _HALF // _L, 1, unroll=16)
        def gather(i):
            sl = pl.ds(i * _L, _L)
            out_v[sl] = plsc.load_gather(slice_v, [idx_v[sl]])

        pltpu.sync_copy(out_v, out_hbm.at[t, pl.ds(h * _HALF, _HALF)])


def _body(table_hbm, idx_hbm, out_hbm, slice_v, idx_v, out_v):
    wid = lax.axis_index("s") * _NC + lax.axis_index("c")

    def round_body(q, _):
        _run_task(table_hbm, idx_hbm, out_hbm, slice_v, idx_v, out_v,
                  q * _NW + wid, (0, 1))
        return _

    lax.fori_loop(0, 6, round_body, None)
    _run_task(table_hbm, idx_hbm, out_hbm, slice_v, idx_v, out_v,
              6 * _NW + lax.rem(wid, 16), (lax.div(wid, 16),))


@functools.partial(
    pl.kernel,
    mesh=plsc.VectorSubcoreMesh(core_axis_name="c", subcore_axis_name="s"),
    out_type=jax.ShapeDtypeStruct((_N_FIELDS * _EDIM, _BATCH), jnp.float32),
    scratch_types=[
        pltpu.VMEM((_VOCAB,), jnp.float32),
        pltpu.VMEM((_HALF,), jnp.int32),
        pltpu.VMEM((_HALF,), jnp.float32),
    ],
    compiler_params=pltpu.CompilerParams(
        use_tc_tiling_on_sc=True, needs_layout_passes=False
    ),
)
def _sc_gather(table_hbm, idx_hbm, out_hbm, slice_v, idx_v, out_v):
    _body(table_hbm, idx_hbm, out_hbm, slice_v, idx_v, out_v)


def kernel(inputs, tables):
    table_t = jnp.transpose(tables, (0, 2, 1))
    idx_t = inputs.T
    out_t = _sc_gather(table_t, idx_t)
    return out_t.T


# speedup vs baseline: 1.1797x; 1.0788x over previous
import functools

import jax
import jax.numpy as jnp
from jax import lax
from jax.experimental import pallas as pl
from jax.experimental.pallas import tpu as pltpu
from jax.experimental.pallas import tpu_sc as plsc

_N_FIELDS = 26
_VOCAB = 100000
_EDIM = 8
_BATCH = 16384

_INFO = plsc.get_sparse_core_info()
_NC = _INFO.num_cores
_NS = _INFO.num_subcores
_NW = _NC * _NS
_L = _INFO.num_lanes

_TASKS = _N_FIELDS * _EDIM
_HALF = _BATCH // 2


def _run_task(refs, t, halves, out_cps):
    table_hbm, idx_hbm, out_hbm, slice_v, idx_v, outs, sem_s, sems_o = refs
    f = t // _EDIM
    d = t % _EDIM
    slice_cp = pltpu.async_copy(table_hbm.at[f, d], slice_v, sem_s)
    first = True
    for h in halves:
        b = h if len(halves) == 2 else 0
        pltpu.sync_copy(idx_hbm.at[f, pl.ds(h * _HALF, _HALF)], idx_v)
        if first:
            slice_cp.wait()
            first = False
        if out_cps[b] is not None:
            out_cps[b].wait()
        out_v = outs[b]

        @plsc.parallel_loop(0, _HALF // _L, 1, unroll=8)
        def gather(i):
            sl = pl.ds(i * _L, _L)
            out_v[sl] = plsc.load_gather(slice_v, [idx_v[sl]])

        out_cps[b] = pltpu.async_copy(
            out_v, out_hbm.at[t, pl.ds(h * _HALF, _HALF)], sems_o[b]
        )


def _body(table_hbm, idx_hbm, out_hbm, slice_v, idx_v, out0_v, out1_v,
          sem_s, sem_o0, sem_o1):
    wid = lax.axis_index("s") * _NC + lax.axis_index("c")
    refs = (table_hbm, idx_hbm, out_hbm, slice_v, idx_v,
            (out0_v, out1_v), sem_s, (sem_o0, sem_o1))
    out_cps = [None, None]

    for q in range(6):
        _run_task(refs, q * _NW + wid, (0, 1), out_cps)
    _run_task(refs, 6 * _NW + lax.rem(wid, 16), (lax.div(wid, 16),), out_cps)
    for cp in out_cps:
        if cp is not None:
            cp.wait()


@functools.partial(
    pl.kernel,
    mesh=plsc.VectorSubcoreMesh(core_axis_name="c", subcore_axis_name="s"),
    out_type=jax.ShapeDtypeStruct((_N_FIELDS * _EDIM, _BATCH), jnp.float32),
    scratch_types=[
        pltpu.VMEM((_VOCAB,), jnp.float32),
        pltpu.VMEM((_HALF,), jnp.int32),
        pltpu.VMEM((_HALF,), jnp.float32),
        pltpu.VMEM((_HALF,), jnp.float32),
        pltpu.SemaphoreType.DMA,
        pltpu.SemaphoreType.DMA,
        pltpu.SemaphoreType.DMA,
    ],
    compiler_params=pltpu.CompilerParams(
        use_tc_tiling_on_sc=True, needs_layout_passes=False
    ),
)
def _sc_gather(table_hbm, idx_hbm, out_hbm, slice_v, idx_v, out0_v, out1_v,
               sem_s, sem_o0, sem_o1):
    _body(table_hbm, idx_hbm, out_hbm, slice_v, idx_v, out0_v, out1_v,
          sem_s, sem_o0, sem_o1)


def kernel(inputs, tables):
    table_t = jnp.transpose(tables, (0, 2, 1))
    idx_t = inputs.T
    out_t = _sc_gather(table_t, idx_t)
    return out_t.T
